# baseline (speedup 1.0000x reference)
import functools

import numpy as np
import jax
import jax.numpy as jnp
from jax import lax
from jax.experimental import pallas as pl
from jax.experimental.pallas import tpu as pltpu
from jax.experimental.pallas import tpu_sc as plsc

_NUM_QUERIES = 900
_NUM_CLASSES = 365
_D = 256
_B = 64
_G = 500
_N = _B * _G
_LABEL_NOISE_PROB = 0.5

_UNITS = 250
_UNITS_LO = _UNITS // 32
_EXTRA = _UNITS - 32 * _UNITS_LO


def _tf_rotl(x, r):
    return (x << np.uint32(r)) | (x >> np.uint32(32 - r))


def _threefry2x32(ks0, ks1, x0, x1):
    rotations = ((13, 15, 26, 6), (17, 29, 16, 24))
    ks = (np.uint32(ks0), np.uint32(ks1),
          np.uint32(ks0) ^ np.uint32(ks1) ^ np.uint32(0x1BD11BDA))
    x0 = x0 + ks[0]
    x1 = x1 + ks[1]
    for i in range(5):
        for r in rotations[i % 2]:
            x0 = x0 + x1
            x1 = _tf_rotl(x1, r)
            x1 = x1 ^ x0
        x0 = x0 + ks[(i + 1) % 3]
        x1 = x1 + ks[(i + 2) % 3] + np.uint32(i + 1)
    return x0, x1


def _random_bits(key, size):
    o0, o1 = _threefry2x32(key[0], key[1], np.zeros(size, np.uint32),
                           np.arange(size, dtype=np.uint32))
    return o0 ^ o1


def _split_key(key, num):
    o0, o1 = _threefry2x32(key[0], key[1], np.zeros(num, np.uint32),
                           np.arange(num, dtype=np.uint32))
    return np.stack([o0, o1], axis=1)


def _uniform01(key, size):
    bits = _random_bits(key, size)
    f = ((bits >> np.uint32(9)) | np.uint32(0x3F800000)).view(np.float32)
    return np.maximum(np.float32(0.0), f - np.float32(1.0))


def _randint(key, size, span):
    k1, k2 = _split_key(key, 2)
    hi = _random_bits(k1, size)
    lo = _random_bits(k2, size)
    span = np.uint32(span)
    mult = np.uint32((int(2 ** 16 % int(span)) ** 2) % int(span))
    off = ((hi % span) * mult + (lo % span)) % span
    return off.astype(np.int32)


@functools.cache
def _noise_consts():
    kmask, klab, kbox = _split_key(np.array([0, 42], np.uint32), 3)
    mask = _uniform01(kmask, _N) < np.float32(_LABEL_NOISE_PROB)
    rand_labels = _randint(klab, _N, _NUM_CLASSES)
    rand_box = (_uniform01(kbox, _N * 4) * np.float32(2.0)
                - np.float32(1.0)).reshape(_N, 4)
    return (
        mask.astype(np.int32),
        rand_labels.astype(np.int32),
        rand_box.astype(np.float32),
    )


@functools.cache
def _attn_mask_const():
    m = np.zeros((_G + _NUM_QUERIES, _G + _NUM_QUERIES), dtype=bool)
    m[_G:, :_G] = True
    return m


_TGT = _G + _NUM_QUERIES


def _tc_noise_body(bt_ref, rt_ref, obox_ref, attn_ref):
    b = bt_ref[...]
    r = rt_ref[...]
    shifted = jnp.concatenate([b[:, 2:, :], b[:, :2, :]], axis=1)
    cidx = lax.broadcasted_iota(jnp.int32, b.shape, 1)
    diff = jnp.where(cidx < 2, shifted * 0.5, b)
    nb = jnp.clip(b + r * diff, 0.0, 1.0)
    eps = 1e-5
    x1 = jnp.maximum(nb, eps)
    x2 = jnp.maximum(1.0 - nb, eps)
    obox_ref[...] = jnp.log(x1 / x2)
    row = lax.broadcasted_iota(jnp.int32, (_TGT, _TGT), 0)
    col = lax.broadcasted_iota(jnp.int32, (_TGT, _TGT), 1)
    attn_ref[...] = (row >= _G) & (col < _G)


def _tc_noise(boxes_t, rbox_t):
    return pl.pallas_call(
        _tc_noise_body,
        out_shape=(jax.ShapeDtypeStruct((_B, 4, _G), jnp.float32),
                   jax.ShapeDtypeStruct((_TGT, _TGT), jnp.bool_)),
    )(boxes_t, rbox_t)


def _sc_gather(labels_t, packed_t, table):
    info = plsc.get_sparse_core_info()
    nc = info.num_cores

    mesh = plsc.VectorSubcoreMesh(core_axis_name="c", subcore_axis_name="s")

    @functools.partial(
        pl.kernel,
        mesh=mesh,
        out_type=jax.ShapeDtypeStruct((_UNITS, 128, _D), jnp.float32),
        scratch_types=[
            pltpu.VMEM((1024,), jnp.int32),
            pltpu.VMEM((1024,), jnp.int32),
            pltpu.VMEM((128, _D), jnp.float32),
            pltpu.VMEM((128, _D), jnp.float32),
            pltpu.VMEM((128, _D), jnp.float32),
            pltpu.SemaphoreType.DMA,
            pltpu.SemaphoreType.DMA,
            pltpu.SemaphoreType.DMA,
            pltpu.SemaphoreType.DMA,
            pltpu.SemaphoreType.DMA,
            pltpu.SemaphoreType.DMA,
        ],
    )
    def k(lab_hbm, packed_hbm, tab_hbm, out_hbm,
          idx_v, p_v, buf0, buf1, buf2,
          g0, g1, g2, s0, s1, s2):
        wid = lax.axis_index("s") * nc + lax.axis_index("c")
        u0 = wid * _UNITS_LO + jnp.minimum(wid, _EXTRA)
        has_extra = wid < _EXTRA
        base = u0 * 128
        cp_l = pltpu.async_copy(lab_hbm.at[pl.ds(base, 1024)], idx_v, g0)
        cp_p = pltpu.async_copy(packed_hbm.at[pl.ds(base, 1024)], p_v, g1)
        cp_l.wait()
        cp_p.wait()
        for i in range(64):
            sl = pl.ds(i * 16, 16)
            p = p_v[sl]
            idx_v[sl] = jnp.where(p >= 0, p, idx_v[sl])
        bufs = (buf0, buf1, buf2)
        gsems = (g0, g1, g2)
        ssems = (s0, s1, s2)
        nu = _UNITS_LO + 1

        def fire(u):
            return pltpu.async_copy(
                tab_hbm.at[idx_v.at[pl.ds(u * 128, 128)]],
                bufs[u % 3], gsems[u % 3])

        gathers = {0: fire(0), 1: fire(1)}
        stores = {}
        for u in range(nu):
            if u + 2 < nu:
                if u - 1 >= 0:
                    stores[u - 1].wait()
                gathers[u + 2] = fire(u + 2)
            gathers[u].wait()
            if u + 1 < nu:
                stores[u] = pltpu.async_copy(bufs[u % 3], out_hbm.at[u0 + u],
                                             ssems[u % 3])
            else:
                @pl.when(has_extra)
                def _():
                    pltpu.async_copy(bufs[u % 3], out_hbm.at[u0 + u],
                                     ssems[u % 3]).wait()
        stores[nu - 3].wait()
        stores[nu - 2].wait()

    return k(labels_t, packed_t, table)


def kernel(gt_labels_list, gt_boxes_list, label_encoder_weight):
    mask_np, rlab_np, rbox_np = _noise_consts()
    boxes_t = jnp.transpose(gt_boxes_list.astype(jnp.float32), (0, 2, 1))
    rbox_t = rbox_np.reshape(_B, _G, 4).transpose(0, 2, 1)
    obox_t, attn_mask = _tc_noise(boxes_t, jnp.asarray(rbox_t))
    lab_t = jnp.pad(
        gt_labels_list.astype(jnp.int32).transpose(1, 0).reshape(-1),
        (0, 768))
    packed = np.where(mask_np != 0, rlab_np, -1).astype(np.int32)
    packed_t = np.pad(packed.reshape(_B, _G).transpose(1, 0).reshape(-1),
                      (0, 768), constant_values=-1)
    plane_out = _sc_gather(lab_t, jnp.asarray(packed_t),
                           label_encoder_weight.astype(jnp.float32))
    noised_label_queries = jnp.transpose(
        plane_out.reshape(_G, _B, _D), (1, 0, 2))
    noised_box_queries = jnp.transpose(obox_t, (0, 2, 1))
    return (noised_label_queries, noised_box_queries, attn_mask, 1, _G)


# speedup vs baseline: 1.0711x; 1.0711x over previous
import functools

import numpy as np
import jax
import jax.numpy as jnp
from jax import lax
from jax.experimental import pallas as pl
from jax.experimental.pallas import tpu as pltpu
from jax.experimental.pallas import tpu_sc as plsc

_NUM_QUERIES = 900
_NUM_CLASSES = 365
_D = 256
_B = 64
_G = 500
_N = _B * _G
_LABEL_NOISE_PROB = 0.5

_PLANES_LO = _G // 32
_PEXTRA = _G - 32 * _PLANES_LO


def _tf_rotl(x, r):
    return (x << np.uint32(r)) | (x >> np.uint32(32 - r))


def _threefry2x32(ks0, ks1, x0, x1):
    rotations = ((13, 15, 26, 6), (17, 29, 16, 24))
    ks = (np.uint32(ks0), np.uint32(ks1),
          np.uint32(ks0) ^ np.uint32(ks1) ^ np.uint32(0x1BD11BDA))
    x0 = x0 + ks[0]
    x1 = x1 + ks[1]
    for i in range(5):
        for r in rotations[i % 2]:
            x0 = x0 + x1
            x1 = _tf_rotl(x1, r)
            x1 = x1 ^ x0
        x0 = x0 + ks[(i + 1) % 3]
        x1 = x1 + ks[(i + 2) % 3] + np.uint32(i + 1)
    return x0, x1


def _random_bits(key, size):
    o0, o1 = _threefry2x32(key[0], key[1], np.zeros(size, np.uint32),
                           np.arange(size, dtype=np.uint32))
    return o0 ^ o1


def _split_key(key, num):
    o0, o1 = _threefry2x32(key[0], key[1], np.zeros(num, np.uint32),
                           np.arange(num, dtype=np.uint32))
    return np.stack([o0, o1], axis=1)


def _uniform01(key, size):
    bits = _random_bits(key, size)
    f = ((bits >> np.uint32(9)) | np.uint32(0x3F800000)).view(np.float32)
    return np.maximum(np.float32(0.0), f - np.float32(1.0))


def _randint(key, size, span):
    k1, k2 = _split_key(key, 2)
    hi = _random_bits(k1, size)
    lo = _random_bits(k2, size)
    span = np.uint32(span)
    mult = np.uint32((int(2 ** 16 % int(span)) ** 2) % int(span))
    off = ((hi % span) * mult + (lo % span)) % span
    return off.astype(np.int32)


@functools.cache
def _noise_consts():
    kmask, klab, kbox = _split_key(np.array([0, 42], np.uint32), 3)
    mask = _uniform01(kmask, _N) < np.float32(_LABEL_NOISE_PROB)
    rand_labels = _randint(klab, _N, _NUM_CLASSES)
    rand_box = (_uniform01(kbox, _N * 4) * np.float32(2.0)
                - np.float32(1.0)).reshape(_N, 4)
    return (
        mask.astype(np.int32),
        rand_labels.astype(np.int32),
        rand_box.astype(np.float32),
    )


@functools.cache
def _attn_mask_const():
    m = np.zeros((_G + _NUM_QUERIES, _G + _NUM_QUERIES), dtype=bool)
    m[_G:, :_G] = True
    return m


def _tc_noise_body(bt_ref, rt_ref, obox_ref):
    b = bt_ref[...]
    r = rt_ref[...]
    shifted = jnp.concatenate([b[:, 2:, :], b[:, :2, :]], axis=1)
    cidx = lax.broadcasted_iota(jnp.int32, b.shape, 1)
    diff = jnp.where(cidx < 2, shifted * 0.5, b)
    nb = jnp.clip(b + r * diff, 0.0, 1.0)
    eps = 1e-5
    x1 = jnp.maximum(nb, eps)
    x2 = jnp.maximum(1.0 - nb, eps)
    obox_ref[...] = jnp.log(x1 / x2)


def _tc_noise(boxes_t, rbox_t):
    return pl.pallas_call(
        _tc_noise_body,
        out_shape=jax.ShapeDtypeStruct((_B, 4, _G), jnp.float32),
    )(boxes_t, rbox_t)


def _sc_gather(labels_t, packed_t, table):
    info = plsc.get_sparse_core_info()
    nc = info.num_cores

    mesh = plsc.VectorSubcoreMesh(core_axis_name="c", subcore_axis_name="s")

    @functools.partial(
        pl.kernel,
        mesh=mesh,
        out_type=jax.ShapeDtypeStruct((_G, _B, _D), jnp.float32),
        scratch_types=[
            pltpu.VMEM((1024,), jnp.int32),
            pltpu.VMEM((1024,), jnp.int32),
            pltpu.VMEM((_B, _D), jnp.float32),
            pltpu.VMEM((_B, _D), jnp.float32),
            pltpu.VMEM((_B, _D), jnp.float32),
            pltpu.VMEM((_B, _D), jnp.float32),
            pltpu.SemaphoreType.DMA,
            pltpu.SemaphoreType.DMA,
            pltpu.SemaphoreType.DMA,
            pltpu.SemaphoreType.DMA,
            pltpu.SemaphoreType.DMA,
            pltpu.SemaphoreType.DMA,
            pltpu.SemaphoreType.DMA,
            pltpu.SemaphoreType.DMA,
        ],
    )
    def k(lab_hbm, packed_hbm, tab_hbm, out_hbm,
          idx_v, p_v, buf0, buf1, buf2, buf3,
          g0, g1, g2, g3, s0, s1, s2, s3):
        wid = lax.axis_index("s") * nc + lax.axis_index("c")
        p0 = wid * _PLANES_LO + jnp.minimum(wid, _PEXTRA)
        has_extra = wid < _PEXTRA
        base = p0 * 64
        cp_l = pltpu.async_copy(lab_hbm.at[pl.ds(base, 1024)], idx_v, g0)
        cp_p = pltpu.async_copy(packed_hbm.at[pl.ds(base, 1024)], p_v, g1)
        cp_l.wait()
        cp_p.wait()
        for i in range(64):
            sl = pl.ds(i * 16, 16)
            p = p_v[sl]
            idx_v[sl] = jnp.where(p >= 0, p, idx_v[sl])
        bufs = (buf0, buf1, buf2, buf3)
        gsems = (g0, g1, g2, g3)
        ssems = (s0, s1, s2, s3)
        nu = _PLANES_LO + 1

        def fire(u):
            return pltpu.async_copy(
                tab_hbm.at[idx_v.at[pl.ds(u * 64, 64)]],
                bufs[u % 4], gsems[u % 4])

        gathers = {0: fire(0), 1: fire(1), 2: fire(2)}
        stores = {}
        for u in range(nu):
            if u + 3 < nu:
                if u - 1 >= 0:
                    stores[u - 1].wait()
                gathers[u + 3] = fire(u + 3)
            gathers[u].wait()
            if u + 1 < nu:
                stores[u] = pltpu.async_copy(bufs[u % 4], out_hbm.at[p0 + u],
                                             ssems[u % 4])
            else:
                @pl.when(has_extra)
                def _():
                    pltpu.async_copy(bufs[u % 4], out_hbm.at[p0 + u],
                                     ssems[u % 4]).wait()
        stores[nu - 4].wait()
        stores[nu - 3].wait()
        stores[nu - 2].wait()

    return k(labels_t, packed_t, table)


def kernel(gt_labels_list, gt_boxes_list, label_encoder_weight):
    mask_np, rlab_np, rbox_np = _noise_consts()
    boxes_t = jnp.transpose(gt_boxes_list.astype(jnp.float32), (0, 2, 1))
    rbox_t = rbox_np.reshape(_B, _G, 4).transpose(0, 2, 1)
    obox_t = _tc_noise(boxes_t, jnp.asarray(rbox_t))
    lab_t = jnp.pad(
        gt_labels_list.astype(jnp.int32).transpose(1, 0).reshape(-1),
        (0, 768))
    packed = np.where(mask_np != 0, rlab_np, -1).astype(np.int32)
    packed_t = np.pad(packed.reshape(_B, _G).transpose(1, 0).reshape(-1),
                      (0, 768), constant_values=-1)
    plane_out = _sc_gather(lab_t, jnp.asarray(packed_t),
                           label_encoder_weight.astype(jnp.float32))
    noised_label_queries = jnp.transpose(plane_out, (1, 0, 2))
    noised_box_queries = jnp.transpose(obox_t, (0, 2, 1))
    attn_mask = jnp.asarray(_attn_mask_const())
    return (noised_label_queries, noised_box_queries, attn_mask, 1, _G)
